# baseline (speedup 1.0000x reference)
import functools

import jax
import jax.numpy as jnp
from jax import lax
from jax.experimental import pallas as pl
from jax.experimental.pallas import tpu as pltpu
from jax.experimental.pallas import tpu_sc as plsc

NUM_TOTEMS = 1000000
POS_DIM = 16
BATCH = 16384

_NC = 2
_NS = 16
_NW = _NC * _NS
_BPW = BATCH // _NW


def _tile_body(idx_hbm, init_hbm, resid_hbm, out_hbm, idx_v, a_v, b_v,
               sem_a, sem_b):
    wid = lax.axis_index("s") * _NC + lax.axis_index("c")
    base = wid * _BPW
    pltpu.sync_copy(idx_hbm.at[pl.ds(base, _BPW)], idx_v)
    cp_a = pltpu.async_copy(init_hbm.at[idx_v], a_v, sem_a)
    cp_b = pltpu.async_copy(resid_hbm.at[idx_v], b_v, sem_b)
    cp_a.wait()
    cp_b.wait()

    def add_row(i, carry):
        a_v[i] = a_v[i] + b_v[i]
        return carry

    lax.fori_loop(0, _BPW, add_row, 0, unroll=8)
    pltpu.sync_copy(a_v, out_hbm.at[pl.ds(base, _BPW)])


@jax.jit
def _lookup(totem_id, init_totem_pos, totem_pos_residual):
    mesh = plsc.VectorSubcoreMesh(core_axis_name="c", subcore_axis_name="s")
    return pl.kernel(
        _tile_body,
        mesh=mesh,
        compiler_params=pltpu.CompilerParams(use_tc_tiling_on_sc=False),
        out_type=jax.ShapeDtypeStruct((BATCH, POS_DIM), jnp.float32),
        scratch_types=[
            pltpu.VMEM((_BPW,), jnp.int32),
            pltpu.VMEM((_BPW, POS_DIM), jnp.float32),
            pltpu.VMEM((_BPW, POS_DIM), jnp.float32),
            pltpu.SemaphoreType.DMA,
            pltpu.SemaphoreType.DMA,
        ],
    )(totem_id, init_totem_pos, totem_pos_residual)


def kernel(totem_id, init_totem_pos, totem_pos_residual):
    return _lookup(totem_id.astype(jnp.int32), init_totem_pos,
                   totem_pos_residual)
